# baseline (speedup 1.0000x reference)
import functools

import jax
import jax.numpy as jnp
from jax import lax
from jax.experimental import pallas as pl
from jax.experimental.pallas import tpu as pltpu
from jax.experimental.pallas import tpu_sc as plsc

_B = 16384
_D = 128
_NUM = 1000000
_R = 4

_NT = 16
_TPT = _B // _NT

_NW = 32
_RPW = _B // _NW


def _ln_body(x_ref, g_ref, b_ref, o_ref):
    x = x_ref[...]
    mu = jnp.mean(x, axis=-1, keepdims=True)
    xc = x - mu
    var = jnp.mean(xc * xc, axis=-1, keepdims=True)
    o_ref[...] = xc * lax.rsqrt(var + 1e-5) * g_ref[...] + b_ref[...]


def _layer_norm_tc(x, g, b):
    blk = 8192
    return pl.pallas_call(
        _ln_body,
        grid=(_B // blk,),
        in_specs=[
            pl.BlockSpec((blk, _D), lambda i: (i, 0)),
            pl.BlockSpec((1, _D), lambda i: (0, 0)),
            pl.BlockSpec((1, _D), lambda i: (0, 0)),
        ],
        out_specs=pl.BlockSpec((blk, _D), lambda i: (i, 0)),
        out_shape=jax.ShapeDtypeStruct((_B, _D), jnp.float32),
    )(x, g.reshape(1, _D), b.reshape(1, _D))


def _dedup_body(ids_hbm, w_hbm, tbl, ids_v, val_v, w_v, idx_v, sem):
    c = lax.axis_index("c")
    s = lax.axis_index("s")

    @pl.when(c == 0)
    def _work():
        base = s * _TPT
        pltpu.sync_copy(ids_hbm.at[pl.ds(base, _TPT)], ids_v)

        def _init_j(j, carry):
            val_v[pl.ds(j * 16, 16)] = base + j * 16 + lax.iota(jnp.int32, 16)
            return carry

        lax.fori_loop(0, _TPT // 16, _init_j, 0)
        pltpu.sync_copy(val_v, tbl.at[ids_v])
        plsc.subcore_barrier()

        def _round(r, carry):
            pltpu.sync_copy(tbl.at[ids_v], w_v)

            def _cmp_j(j, carry2):
                sl = pl.ds(j * 16, 16)
                idx_v[sl] = jnp.where(
                    w_v[sl] < val_v[sl], ids_v[sl], val_v[sl] + _NUM)
                return carry2

            lax.fori_loop(0, _TPT // 16, _cmp_j, carry)
            pltpu.sync_copy(val_v, tbl.at[idx_v])
            plsc.subcore_barrier()
            return carry

        lax.fori_loop(0, _R, _round, 0)
        pltpu.sync_copy(tbl.at[ids_v], w_v)
        pltpu.sync_copy(w_v, w_hbm.at[pl.ds(base, _TPT)])


def _dedup_sc(node_ids):
    mesh = plsc.VectorSubcoreMesh(core_axis_name="c", subcore_axis_name="s")
    f = functools.partial(
        pl.kernel,
        out_type=jax.ShapeDtypeStruct((_B,), jnp.int32),
        scratch_types=[
            pltpu.VMEM_SHARED((_NUM + _B,), jnp.int32),
            pltpu.VMEM((_TPT,), jnp.int32),
            pltpu.VMEM((_TPT,), jnp.int32),
            pltpu.VMEM((_TPT,), jnp.int32),
            pltpu.VMEM((_TPT,), jnp.int32),
            pltpu.SemaphoreType.DMA,
        ],
        mesh=mesh,
    )(_dedup_body)
    return f(node_ids)


def _gather_body(norm_hbm, widx_hbm, out_hbm, idx_v, rows_v, sem):
    c = lax.axis_index("c")
    s = lax.axis_index("s")
    base = (s * 2 + c) * _RPW
    pltpu.sync_copy(widx_hbm.at[pl.ds(base, _RPW)], idx_v)
    pltpu.async_copy(norm_hbm.at[idx_v], rows_v, sem).wait()
    pltpu.sync_copy(rows_v, out_hbm.at[pl.ds(base, _RPW)])


def _gather_sc(normalized, widx):
    mesh = plsc.VectorSubcoreMesh(core_axis_name="c", subcore_axis_name="s")
    f = functools.partial(
        pl.kernel,
        out_type=jax.ShapeDtypeStruct((_B, _D), jnp.float32),
        scratch_types=[
            pltpu.VMEM((_RPW,), jnp.int32),
            pltpu.VMEM((_RPW, _D), jnp.float32),
            pltpu.SemaphoreType.DMA,
        ],
        mesh=mesh,
    )(_gather_body)
    return f(normalized, widx)


def kernel(node_ids, updated_node_memories, new_times, node_memories,
           node_last_updated_times, ln_weight, ln_bias):
    ids = node_ids.astype(jnp.int32)
    normalized = _layer_norm_tc(updated_node_memories, ln_weight, ln_bias)
    winner = _dedup_sc(ids)
    return _gather_sc(normalized, winner)


# speedup vs baseline: 1.0027x; 1.0027x over previous
import functools

import jax
import jax.numpy as jnp
from jax import lax
from jax.experimental import pallas as pl
from jax.experimental.pallas import tpu as pltpu
from jax.experimental.pallas import tpu_sc as plsc

_B = 16384
_D = 128
_NUM = 1000000
_R = 4

_NT = 16
_TPT = _B // _NT

_NW = 32
_RPW = _B // _NW


def _ln_body(x_ref, g_ref, b_ref, o_ref):
    x = x_ref[...]
    mu = jnp.mean(x, axis=-1, keepdims=True)
    xc = x - mu
    var = jnp.mean(xc * xc, axis=-1, keepdims=True)
    o_ref[...] = xc * lax.rsqrt(var + 1e-5) * g_ref[...] + b_ref[...]


def _layer_norm_tc(x, g, b):
    blk = 8192
    return pl.pallas_call(
        _ln_body,
        grid=(_B // blk,),
        in_specs=[
            pl.BlockSpec((blk, _D), lambda i: (i, 0)),
            pl.BlockSpec((1, _D), lambda i: (0, 0)),
            pl.BlockSpec((1, _D), lambda i: (0, 0)),
        ],
        out_specs=pl.BlockSpec((blk, _D), lambda i: (i, 0)),
        out_shape=jax.ShapeDtypeStruct((_B, _D), jnp.float32),
    )(x, g.reshape(1, _D), b.reshape(1, _D))


def _dedup_body(ids_hbm, w_hbm, tbl, ids_v, val_v, w_v, idx_v, sem):
    c = lax.axis_index("c")
    s = lax.axis_index("s")

    @pl.when(c == 0)
    def _work():
        base = s * _TPT
        pltpu.sync_copy(ids_hbm.at[pl.ds(base, _TPT)], ids_v)

        iota16 = lax.iota(jnp.int32, 16)
        for j in range(_TPT // 16):
            val_v[pl.ds(j * 16, 16)] = base + j * 16 + iota16
        pltpu.sync_copy(val_v, tbl.at[ids_v])
        plsc.subcore_barrier()

        def _round(r, carry):
            pltpu.sync_copy(tbl.at[ids_v], w_v)
            for j in range(_TPT // 16):
                sl = pl.ds(j * 16, 16)
                idx_v[sl] = jnp.where(
                    w_v[sl] < val_v[sl], ids_v[sl], val_v[sl] + _NUM)
            pltpu.sync_copy(val_v, tbl.at[idx_v])
            plsc.subcore_barrier()
            return carry

        lax.fori_loop(0, _R, _round, 0)
        pltpu.sync_copy(tbl.at[ids_v], w_v)
        pltpu.sync_copy(w_v, w_hbm.at[pl.ds(base, _TPT)])


def _dedup_sc(node_ids):
    mesh = plsc.VectorSubcoreMesh(core_axis_name="c", subcore_axis_name="s")
    f = functools.partial(
        pl.kernel,
        out_type=jax.ShapeDtypeStruct((_B,), jnp.int32),
        scratch_types=[
            pltpu.VMEM_SHARED((_NUM + _B,), jnp.int32),
            pltpu.VMEM((_TPT,), jnp.int32),
            pltpu.VMEM((_TPT,), jnp.int32),
            pltpu.VMEM((_TPT,), jnp.int32),
            pltpu.VMEM((_TPT,), jnp.int32),
            pltpu.SemaphoreType.DMA,
        ],
        mesh=mesh,
    )(_dedup_body)
    return f(node_ids)


_HGC = 2
_HRW = _RPW // _HGC


def _gather_body(norm_hbm, widx_hbm, out_hbm, idx_v, rows_v, sems, semw):
    c = lax.axis_index("c")
    s = lax.axis_index("s")
    base = (s * 2 + c) * _RPW
    pltpu.sync_copy(widx_hbm.at[pl.ds(base, _RPW)], idx_v)
    gs = [
        pltpu.async_copy(
            norm_hbm.at[idx_v.at[pl.ds(h * _HRW, _HRW)]],
            rows_v.at[h], sems.at[h])
        for h in range(_HGC)
    ]
    ws = []
    for h in range(_HGC):
        gs[h].wait()
        ws.append(pltpu.async_copy(
            rows_v.at[h], out_hbm.at[pl.ds(base + h * _HRW, _HRW)], semw))
    for cp in ws:
        cp.wait()


def _gather_sc(normalized, widx):
    mesh = plsc.VectorSubcoreMesh(core_axis_name="c", subcore_axis_name="s")
    f = functools.partial(
        pl.kernel,
        out_type=jax.ShapeDtypeStruct((_B, _D), jnp.float32),
        scratch_types=[
            pltpu.VMEM((_RPW,), jnp.int32),
            pltpu.VMEM((_HGC, _HRW, _D), jnp.float32),
            pltpu.SemaphoreType.DMA((_HGC,)),
            pltpu.SemaphoreType.DMA,
        ],
        mesh=mesh,
    )(_gather_body)
    return f(normalized, widx)


def kernel(node_ids, updated_node_memories, new_times, node_memories,
           node_last_updated_times, ln_weight, ln_bias):
    ids = node_ids.astype(jnp.int32)
    normalized = _layer_norm_tc(updated_node_memories, ln_weight, ln_bias)
    winner = _dedup_sc(ids)
    return _gather_sc(normalized, winner)
